# baseline (speedup 1.0000x reference)
import functools

import jax
import jax.numpy as jnp
from jax import lax
from jax.experimental import pallas as pl
from jax.experimental.pallas import tpu as pltpu
from jax.experimental.pallas import tpu_sc as plsc

N = 10000
E = 320000
D = 128
HK = 16
EMB = 64
EPS = 1e-5

NC = 2
NS = 16
NW = NC * NS
CE = 64
EPT = 10240
T0 = 224
T1 = 96
NCHUNK = EPT // CE
EPAD = NW * EPT
NBUF = 4
EPADX = EPAD + 3 * CE
NPAD = 10240
ROWS_PER_TILE = NPAD // NS

RB = 1000
GRID = N // RB


def _edge_body(ep_hbm, a_hbm, b_hbm, x_hbm, cl_hbm,
               out_hbm,
               epk0, epk1, epk2, epk3, ad0, ad1, ad2, ad3,
               bs0, bs1, bs2, bs3, xb0, xb1, xb2, xb3, msg0, msg1, wv, clv,
               aggr, sem_i0, sem_i1, sem_i2, sem_i3,
               sem_g0, sem_g1, sem_g2, sem_g3,
               sem_s0, sem_s1, sem_s2, sem_s3):
    c = lax.axis_index("c")
    s = lax.axis_index("s")
    ebase = jnp.where(c == 0, s * (T0 * CE), NS * (T0 * CE) + s * (T1 * CE))
    nquad = jnp.where(c == 0, T0 // NBUF, T1 // NBUF)

    epk = (epk0, epk1, epk2, epk3)
    ad = (ad0, ad1, ad2, ad3)
    bs = (bs0, bs1, bs2, bs3)
    xb = (xb0, xb1, xb2, xb3)
    msg = (msg0, msg1)
    sem_i = (sem_i0, sem_i1, sem_i2, sem_i3)
    sem_g = (sem_g0, sem_g1, sem_g2, sem_g3)
    sem_s = (sem_s0, sem_s1, sem_s2, sem_s3)

    pltpu.sync_copy(cl_hbm, clv)
    cvec = clv[0, :]
    lvec = clv[1, :]
    lbb_s = clv[2, :][0]

    def _zrow(i, carry):
        for r in range(D // 16):
            msg0[i, pl.ds(r * 16, 16)] = jnp.zeros((16,), jnp.float32)
        return carry
    lax.fori_loop(0, CE, _zrow, 0)

    def _zcp(j, carry):
        pltpu.sync_copy(msg0, aggr.at[pl.ds(s * ROWS_PER_TILE + j * CE, CE)])
        return carry
    lax.fori_loop(0, ROWS_PER_TILE // CE, _zcp, 0)
    plsc.subcore_barrier()

    def _issue_idx(i, b):
        pltpu.async_copy(ep_hbm.at[:, pl.ds(ebase + i * CE, CE)],
                         epk[b], sem_i[b])

    def _wait_idx(b):
        pltpu.make_async_copy(ep_hbm.at[:, pl.ds(ebase, CE)],
                              epk[b], sem_i[b]).wait()

    def _issue_gathers(b):
        pltpu.async_copy(a_hbm.at[epk[b].at[1]], ad[b], sem_g[b])
        pltpu.async_copy(b_hbm.at[epk[b].at[0]], bs[b], sem_g[b])
        pltpu.async_copy(x_hbm.at[epk[b].at[0]], xb[b], sem_g[b])

    def _wait_gathers(b):
        pltpu.make_async_copy(a_hbm.at[epk[b].at[1]], ad[b], sem_g[b]).wait()
        pltpu.make_async_copy(b_hbm.at[epk[b].at[0]], bs[b], sem_g[b]).wait()
        pltpu.make_async_copy(x_hbm.at[epk[b].at[0]], xb[b], sem_g[b]).wait()

    def _wait_scatter(b):
        pltpu.make_async_copy(msg[b % 2], aggr.at[epk[b].at[1]],
                              sem_s[b]).wait()

    _issue_idx(0, 0)
    _issue_idx(1, 1)
    _issue_idx(2, 2)
    _wait_idx(0)
    _issue_gathers(0)
    _wait_idx(1)
    _issue_gathers(1)

    def _quad(j, carry):
        for b in range(NBUF):
            i = j * NBUF + b

            @pl.when(i >= 1)
            def _():
                _wait_scatter((b - 1) % NBUF)

            _issue_idx(i + 3, (b + 3) % NBUF)
            _wait_gathers(b)

            def _grp(g, carry2, b=b):
                o = g * 16
                sv = epk[b][0, pl.ds(o, 16)]
                dv = epk[b][1, pl.ds(o, 16)]
                ev = plsc.bitcast(epk[b][2, pl.ds(o, 16)], jnp.float32)
                rows = lax.iota(jnp.int32, 16) + o
                logit = jnp.zeros((16,), jnp.float32) + lbb_s
                for k in range(HK):
                    colk = jnp.full((16,), k, jnp.int32)
                    hk = (plsc.load_gather(ad[b], [rows, colk])
                          + plsc.load_gather(bs[b], [rows, colk])
                          + ev * cvec[k])
                    hk = jnp.maximum(hk, 0.0)
                    logit = logit + hk * lvec[k]
                w = 1.0 / (1.0 + jnp.exp(-logit))
                w = jnp.where(sv != dv, w, 0.0)
                wv[pl.ds(o, 16)] = w
                return carry2
            lax.fori_loop(0, CE // 16, _grp, 0)

            _wait_idx((b + 2) % NBUF)
            _issue_gathers((b + 2) % NBUF)

            def _scl(g, carry2, b=b):
                o = g * 16
                wvec = wv[pl.ds(o, 16)]
                for e in range(16):
                    w_s = wvec[e]
                    for r in range(D // 32):
                        row32 = xb[b][o + e, pl.ds(r * 32, 32)]
                        lo, hi = plsc.unpack(
                            row32, format=plsc.PackFormat.INTERLEAVED,
                            preferred_element_type=jnp.float32)
                        msg[b % 2][o + e, pl.ds(r * 32, 16)] = lo * w_s
                        msg[b % 2][o + e, pl.ds(r * 32 + 16, 16)] = hi * w_s
                return carry2
            lax.fori_loop(0, CE // 16, _scl, 0)

            pltpu.async_copy(msg[b % 2], aggr.at[epk[b].at[1]], sem_s[b],
                             add=True)
        return carry
    lax.fori_loop(0, nquad, _quad, 0)

    _wait_scatter(NBUF - 1)
    _wait_gathers(0)
    _wait_gathers(1)
    _wait_idx(2)

    plsc.subcore_barrier()
    pltpu.sync_copy(aggr.at[pl.ds(s * ROWS_PER_TILE, ROWS_PER_TILE)],
                    out_hbm.at[c, pl.ds(s * ROWS_PER_TILE, ROWS_PER_TILE)])


_edge_aggregate = functools.partial(
    pl.kernel,
    out_type=jax.ShapeDtypeStruct((NC, NPAD, D), jnp.float32),
    mesh=plsc.VectorSubcoreMesh(core_axis_name="c", subcore_axis_name="s"),
    compiler_params=pltpu.CompilerParams(needs_layout_passes=False,
                                         use_tc_tiling_on_sc=False),
    scratch_types=(
        [pltpu.VMEM((3, CE), jnp.int32)] * NBUF
        + [pltpu.VMEM((CE, HK), jnp.float32)] * NBUF
        + [pltpu.VMEM((CE, HK), jnp.float32)] * NBUF
        + [pltpu.VMEM((CE, D), jnp.bfloat16)] * NBUF
        + [pltpu.VMEM((CE, D), jnp.float32)] * 2
        + [
            pltpu.VMEM((CE,), jnp.float32),
            pltpu.VMEM((3, HK), jnp.float32),
            pltpu.VMEM_SHARED((NPAD, D), jnp.float32),
        ]
        + [pltpu.SemaphoreType.DMA] * 12
    ),
)(_edge_body)


def _dot(a, b):
    return jnp.dot(a, b, preferred_element_type=jnp.float32)


def _proj_body(x_ref, wd_ref, ws_ref, lab_ref, a_ref, b_ref):
    xb = x_ref[...]
    a_ref[...] = _dot(xb, wd_ref[...]) + lab_ref[...]
    b_ref[...] = _dot(xb, ws_ref[...])


def _proj(x, wdT, wsT, lab):
    return pl.pallas_call(
        _proj_body,
        grid=(GRID,),
        in_specs=[
            pl.BlockSpec((RB, D), lambda i: (i, 0)),
            pl.BlockSpec((D, HK), lambda i: (0, 0)),
            pl.BlockSpec((D, HK), lambda i: (0, 0)),
            pl.BlockSpec((1, HK), lambda i: (0, 0)),
        ],
        out_specs=[
            pl.BlockSpec((RB, HK), lambda i: (i, 0)),
            pl.BlockSpec((RB, HK), lambda i: (i, 0)),
        ],
        out_shape=[
            jax.ShapeDtypeStruct((N, HK), jnp.float32),
            jax.ShapeDtypeStruct((N, HK), jnp.float32),
        ],
    )(x, wdT, wsT, lab)


def _self_gate(xb, wsumT, lab, lb, lbb):
    hs = jnp.maximum(_dot(xb, wsumT) + lab, 0.0)
    logit = jnp.sum(hs * lb, axis=1, keepdims=True) + lbb
    return (1.0 / (1.0 + jnp.exp(-logit))) * xb


def _combine0_body(x_ref, agg_ref, wsum_ref, lab_ref, lb_ref, lbb_ref,
                   ua_ref, ub_ref, b_ref, emb_ref, sums_ref):
    xb = x_ref[...]
    selfm = _self_gate(xb, wsum_ref[...], lab_ref[...], lb_ref[...],
                       lbb_ref[...])
    aggr = agg_ref[0] + agg_ref[1] + selfm
    e = jnp.maximum(_dot(xb, ua_ref[...]) + _dot(aggr, ub_ref[...])
                    + b_ref[...], 0.0)
    emb_ref[...] = e

    @pl.when(pl.program_id(0) == 0)
    def _():
        sums_ref[...] = jnp.zeros_like(sums_ref)
    sums_ref[...] += jnp.concatenate(
        [jnp.sum(e, axis=0, keepdims=True),
         jnp.sum(e * e, axis=0, keepdims=True)], axis=0)


def _combine0(x, agg, wsumT, lab, lb, lbb, ua, ub, b):
    return pl.pallas_call(
        _combine0_body,
        grid=(GRID,),
        in_specs=[
            pl.BlockSpec((RB, D), lambda i: (i, 0)),
            pl.BlockSpec((NC, RB, D), lambda i: (0, i, 0)),
            pl.BlockSpec((D, HK), lambda i: (0, 0)),
            pl.BlockSpec((1, HK), lambda i: (0, 0)),
            pl.BlockSpec((1, HK), lambda i: (0, 0)),
            pl.BlockSpec((1, 1), lambda i: (0, 0)),
            pl.BlockSpec((D, D), lambda i: (0, 0)),
            pl.BlockSpec((D, D), lambda i: (0, 0)),
            pl.BlockSpec((1, D), lambda i: (0, 0)),
        ],
        out_specs=[
            pl.BlockSpec((RB, D), lambda i: (i, 0)),
            pl.BlockSpec((2, D), lambda i: (0, 0)),
        ],
        out_shape=[
            jax.ShapeDtypeStruct((N, D), jnp.float32),
            jax.ShapeDtypeStruct((2, D), jnp.float32),
        ],
    )(x, agg, wsumT, lab, lb, lbb, ua, ub, b)


def _normproj_body(emb_ref, sums_ref, gamma_ref, beta_ref,
                   wd_ref, ws_ref, lab_ref, xn_ref, a_ref, b_ref):
    sums = sums_ref[...]
    mean = sums[0:1] * (1.0 / N)
    var = sums[1:2] * (1.0 / N) - mean * mean
    xb = emb_ref[...]
    xn = ((xb - mean) * lax.rsqrt(var + EPS) * gamma_ref[...]
          + beta_ref[...])
    xn_ref[...] = xn
    a_ref[...] = _dot(xn, wd_ref[...]) + lab_ref[...]
    b_ref[...] = _dot(xn, ws_ref[...])


def _normproj(emb, sums, gamma, beta, wdT, wsT, lab):
    return pl.pallas_call(
        _normproj_body,
        grid=(GRID,),
        in_specs=[
            pl.BlockSpec((RB, D), lambda i: (i, 0)),
            pl.BlockSpec((2, D), lambda i: (0, 0)),
            pl.BlockSpec((1, D), lambda i: (0, 0)),
            pl.BlockSpec((1, D), lambda i: (0, 0)),
            pl.BlockSpec((D, HK), lambda i: (0, 0)),
            pl.BlockSpec((D, HK), lambda i: (0, 0)),
            pl.BlockSpec((1, HK), lambda i: (0, 0)),
        ],
        out_specs=[
            pl.BlockSpec((RB, D), lambda i: (i, 0)),
            pl.BlockSpec((RB, HK), lambda i: (i, 0)),
            pl.BlockSpec((RB, HK), lambda i: (i, 0)),
        ],
        out_shape=[
            jax.ShapeDtypeStruct((N, D), jnp.float32),
            jax.ShapeDtypeStruct((N, HK), jnp.float32),
            jax.ShapeDtypeStruct((N, HK), jnp.float32),
        ],
    )(emb, sums, gamma, beta, wdT, wsT, lab)


def _combine1_body(xn_ref, agg_ref, wsum_ref, lab_ref, lb_ref, lbb_ref,
                   ua_ref, ub_ref, b_ref, few_ref, feb_ref, out_ref):
    xb = xn_ref[...]
    selfm = _self_gate(xb, wsum_ref[...], lab_ref[...], lb_ref[...],
                       lbb_ref[...])
    aggr = agg_ref[0] + agg_ref[1] + selfm
    e = jnp.maximum(_dot(xb, ua_ref[...]) + _dot(aggr, ub_ref[...])
                    + b_ref[...], 0.0)
    out_ref[...] = _dot(e, few_ref[...]) + feb_ref[...]


def _combine1(xn, agg, wsumT, lab, lb, lbb, ua, ub, b, fewT, feb):
    return pl.pallas_call(
        _combine1_body,
        grid=(GRID,),
        in_specs=[
            pl.BlockSpec((RB, D), lambda i: (i, 0)),
            pl.BlockSpec((NC, RB, D), lambda i: (0, i, 0)),
            pl.BlockSpec((D, HK), lambda i: (0, 0)),
            pl.BlockSpec((1, HK), lambda i: (0, 0)),
            pl.BlockSpec((1, HK), lambda i: (0, 0)),
            pl.BlockSpec((1, 1), lambda i: (0, 0)),
            pl.BlockSpec((D, D), lambda i: (0, 0)),
            pl.BlockSpec((D, D), lambda i: (0, 0)),
            pl.BlockSpec((1, D), lambda i: (0, 0)),
            pl.BlockSpec((D, EMB), lambda i: (0, 0)),
            pl.BlockSpec((1, EMB), lambda i: (0, 0)),
        ],
        out_specs=pl.BlockSpec((RB, EMB), lambda i: (i, 0)),
        out_shape=jax.ShapeDtypeStruct((N, EMB), jnp.float32),
    )(xn, agg, wsumT, lab, lb, lbb, ua, ub, b, fewT, feb)


def kernel(x, edge_index, edge_attr,
           l0_lin_W, l0_lin_b, l0_linA_W, l0_linA_b, l0_linB_W, l0_linB_b,
           l1_bn_gamma, l1_bn_beta, l1_lin_W, l1_lin_b, l1_linA_W, l1_linA_b,
           l1_linB_W, l1_linB_b, fe_W, fe_b):
    f32 = jnp.float32
    src = edge_index[0].astype(jnp.int32)
    dst = edge_index[1].astype(jnp.int32)
    ea = edge_attr.astype(f32)
    pad = EPADX - E
    zpad_i = jnp.zeros((pad,), jnp.int32)
    epack = jnp.stack([
        jnp.concatenate([src, zpad_i]),
        jnp.concatenate([dst, zpad_i]),
        jnp.concatenate([lax.bitcast_convert_type(ea, jnp.int32), zpad_i]),
    ], axis=0)

    def prep(laW, lab, lbW, lbb):
        wdT = laW[:, :D].T
        wsT = laW[:, D:2 * D].T
        wsumT = wdT + wsT
        cvec = laW[:, 2 * D]
        cl = jnp.stack([cvec, lbW[0],
                        jnp.full((HK,), lbb[0], f32)], axis=0)
        lab2 = lab.reshape(1, HK)
        lb2 = lbW.reshape(1, HK)
        lbb2 = lbb.reshape(1, 1)
        return wdT, wsT, wsumT, cl, lab2, lb2, lbb2

    def ileave(xf):
        xr = xf.reshape(N, D // 32, 2, 16)
        return xr.swapaxes(2, 3).reshape(N, D).astype(jnp.bfloat16)

    wdT0, wsT0, wsumT0, cl0, lab0, lb0, lbb0 = prep(
        l0_linA_W, l0_linA_b, l0_linB_W, l0_linB_b)
    wdT1, wsT1, wsumT1, cl1, lab1, lb1, lbb1 = prep(
        l1_linA_W, l1_linA_b, l1_linB_W, l1_linB_b)

    ua0 = l0_lin_W[:, :D].T
    ub0 = l0_lin_W[:, D:].T
    b0 = l0_lin_b.reshape(1, D)
    ua1 = l1_lin_W[:, :D].T
    ub1 = l1_lin_W[:, D:].T
    b1 = l1_lin_b.reshape(1, D)
    fewT = fe_W.T
    feb2 = fe_b.reshape(1, EMB)
    gamma = l1_bn_gamma.reshape(1, D)
    beta = l1_bn_beta.reshape(1, D)

    a0, bvec0 = _proj(x, wdT0, wsT0, lab0)
    agg0 = _edge_aggregate(epack, a0, bvec0, ileave(x), cl0)
    emb0, sums = _combine0(x, agg0, wsumT0, lab0, lb0, lbb0, ua0, ub0, b0)

    xn, a1, bvec1 = _normproj(emb0, sums, gamma, beta, wdT1, wsT1, lab1)
    agg1 = _edge_aggregate(epack, a1, bvec1, ileave(xn), cl1)
    return _combine1(xn, agg1, wsumT1, lab1, lb1, lbb1, ua1, ub1, b1,
                     fewT, feb2)


# speedup vs baseline: 1.1983x; 1.1983x over previous
import functools

import jax
import jax.numpy as jnp
from jax import lax
from jax.experimental import pallas as pl
from jax.experimental.pallas import tpu as pltpu
from jax.experimental.pallas import tpu_sc as plsc

N = 10000
E = 320000
D = 128
HK = 16
EMB = 64
EPS = 1e-5

NC = 2
NS = 16
NW = NC * NS
CE = 64
EPT = 10240
T0 = 200
T1 = 120
NCHUNK = EPT // CE
EPAD = NW * EPT
NBUF = 4
EPADX = EPAD + 3 * CE
NPAD = 10240
ROWS_PER_TILE = NPAD // NS

RB = 1000
GRID = N // RB


def _edge_body(ep_hbm, a_hbm, b_hbm, x_hbm, cl_hbm,
               out_hbm,
               epk0, epk1, epk2, epk3, ad0, ad1, ad2, ad3,
               bs0, bs1, bs2, bs3, xb0, xb1, xb2, xb3, wv, clv,
               aggr, sem_i0, sem_i1, sem_i2, sem_i3,
               sem_g0, sem_g1, sem_g2, sem_g3,
               sem_s0, sem_s1, sem_s2, sem_s3):
    c = lax.axis_index("c")
    s = lax.axis_index("s")
    ebase = jnp.where(c == 0, s * (T0 * CE), NS * (T0 * CE) + s * (T1 * CE))
    nquad = jnp.where(c == 0, T0 // NBUF, T1 // NBUF)

    epk = (epk0, epk1, epk2, epk3)
    ad = (ad0, ad1, ad2, ad3)
    bs = (bs0, bs1, bs2, bs3)
    xb = (xb0, xb1, xb2, xb3)
    sem_i = (sem_i0, sem_i1, sem_i2, sem_i3)
    sem_g = (sem_g0, sem_g1, sem_g2, sem_g3)
    sem_s = (sem_s0, sem_s1, sem_s2, sem_s3)

    pltpu.sync_copy(cl_hbm, clv)
    cvec = clv[0, :]
    lvec = clv[1, :]
    lbb_s = clv[2, :][0]

    def _zrow(i, carry):
        for r in range(D // 16):
            xb0[i, pl.ds(r * 16, 16)] = jnp.zeros((16,), jnp.float32)
        return carry
    lax.fori_loop(0, CE, _zrow, 0)

    def _zcp(j, carry):
        pltpu.sync_copy(xb0, aggr.at[pl.ds(s * ROWS_PER_TILE + j * CE, CE)])
        return carry
    lax.fori_loop(0, ROWS_PER_TILE // CE, _zcp, 0)
    plsc.subcore_barrier()

    def _issue_idx(i, b):
        pltpu.async_copy(ep_hbm.at[:, pl.ds(ebase + i * CE, CE)],
                         epk[b], sem_i[b])

    def _wait_idx(b):
        pltpu.make_async_copy(ep_hbm.at[:, pl.ds(ebase, CE)],
                              epk[b], sem_i[b]).wait()

    def _issue_gathers(b):
        pltpu.async_copy(a_hbm.at[epk[b].at[1]], ad[b], sem_g[b])
        pltpu.async_copy(b_hbm.at[epk[b].at[0]], bs[b], sem_g[b])
        pltpu.async_copy(x_hbm.at[epk[b].at[0]], xb[b], sem_g[b])

    def _wait_gathers(b):
        pltpu.make_async_copy(a_hbm.at[epk[b].at[1]], ad[b], sem_g[b]).wait()
        pltpu.make_async_copy(b_hbm.at[epk[b].at[0]], bs[b], sem_g[b]).wait()
        pltpu.make_async_copy(x_hbm.at[epk[b].at[0]], xb[b], sem_g[b]).wait()

    def _wait_scatter(b):
        pltpu.make_async_copy(xb[b], aggr.at[epk[b].at[1]], sem_s[b]).wait()

    _issue_idx(0, 0)
    _issue_idx(1, 1)
    _issue_idx(2, 2)
    _wait_idx(0)
    _issue_gathers(0)
    _wait_idx(1)
    _issue_gathers(1)

    def _quad(j, carry):
        for b in range(NBUF):
            i = j * NBUF + b

            @pl.when(i >= 1)
            def _():
                _wait_scatter((b - 1) % NBUF)

            _issue_idx(i + 3, (b + 3) % NBUF)
            _wait_gathers(b)

            def _grp(g, carry2, b=b):
                o = g * 16
                sv = epk[b][0, pl.ds(o, 16)]
                dv = epk[b][1, pl.ds(o, 16)]
                ev = plsc.bitcast(epk[b][2, pl.ds(o, 16)], jnp.float32)
                rows = lax.iota(jnp.int32, 16) + o
                logit = jnp.zeros((16,), jnp.float32) + lbb_s
                for k in range(HK):
                    colk = jnp.full((16,), k, jnp.int32)
                    hk = (plsc.load_gather(ad[b], [rows, colk])
                          + plsc.load_gather(bs[b], [rows, colk])
                          + ev * cvec[k])
                    hk = jnp.maximum(hk, 0.0)
                    logit = logit + hk * lvec[k]
                w = 1.0 / (1.0 + jnp.exp(-logit))
                w = jnp.where(sv != dv, w, 0.0)
                wv[pl.ds(o, 16)] = w
                return carry2
            lax.fori_loop(0, CE // 16, _grp, 0)

            _wait_idx((b + 2) % NBUF)
            _issue_gathers((b + 2) % NBUF)

            def _scl(g, carry2, b=b):
                o = g * 16
                wvec = wv[pl.ds(o, 16)]
                for e in range(16):
                    w_s = wvec[e]
                    for r in range(D // 16):
                        xb[b][o + e, pl.ds(r * 16, 16)] = (
                            xb[b][o + e, pl.ds(r * 16, 16)] * w_s)
                return carry2
            lax.fori_loop(0, CE // 16, _scl, 0)

            pltpu.async_copy(xb[b], aggr.at[epk[b].at[1]], sem_s[b],
                             add=True)
        return carry
    lax.fori_loop(0, nquad, _quad, 0)

    _wait_scatter(NBUF - 1)
    _wait_gathers(0)
    _wait_gathers(1)
    _wait_idx(2)

    plsc.subcore_barrier()
    pltpu.sync_copy(aggr.at[pl.ds(s * ROWS_PER_TILE, ROWS_PER_TILE)],
                    out_hbm.at[c, pl.ds(s * ROWS_PER_TILE, ROWS_PER_TILE)])


_edge_aggregate = functools.partial(
    pl.kernel,
    out_type=jax.ShapeDtypeStruct((NC, NPAD, D), jnp.float32),
    mesh=plsc.VectorSubcoreMesh(core_axis_name="c", subcore_axis_name="s"),
    compiler_params=pltpu.CompilerParams(needs_layout_passes=False,
                                         use_tc_tiling_on_sc=False),
    scratch_types=(
        [pltpu.VMEM((3, CE), jnp.int32)] * NBUF
        + [pltpu.VMEM((CE, HK), jnp.float32)] * NBUF
        + [pltpu.VMEM((CE, HK), jnp.float32)] * NBUF
        + [pltpu.VMEM((CE, D), jnp.float32)] * NBUF
        + [
            pltpu.VMEM((CE,), jnp.float32),
            pltpu.VMEM((3, HK), jnp.float32),
            pltpu.VMEM_SHARED((NPAD, D), jnp.float32),
        ]
        + [pltpu.SemaphoreType.DMA] * 12
    ),
)(_edge_body)


def _dot(a, b):
    return jnp.dot(a, b, preferred_element_type=jnp.float32)


def _proj_body(x_ref, wd_ref, ws_ref, lab_ref, a_ref, b_ref):
    xb = x_ref[...]
    a_ref[...] = _dot(xb, wd_ref[...]) + lab_ref[...]
    b_ref[...] = _dot(xb, ws_ref[...])


def _proj(x, wdT, wsT, lab):
    return pl.pallas_call(
        _proj_body,
        grid=(GRID,),
        in_specs=[
            pl.BlockSpec((RB, D), lambda i: (i, 0)),
            pl.BlockSpec((D, HK), lambda i: (0, 0)),
            pl.BlockSpec((D, HK), lambda i: (0, 0)),
            pl.BlockSpec((1, HK), lambda i: (0, 0)),
        ],
        out_specs=[
            pl.BlockSpec((RB, HK), lambda i: (i, 0)),
            pl.BlockSpec((RB, HK), lambda i: (i, 0)),
        ],
        out_shape=[
            jax.ShapeDtypeStruct((N, HK), jnp.float32),
            jax.ShapeDtypeStruct((N, HK), jnp.float32),
        ],
    )(x, wdT, wsT, lab)


def _self_gate(xb, wsumT, lab, lb, lbb):
    hs = jnp.maximum(_dot(xb, wsumT) + lab, 0.0)
    logit = jnp.sum(hs * lb, axis=1, keepdims=True) + lbb
    return (1.0 / (1.0 + jnp.exp(-logit))) * xb


def _combine0_body(x_ref, agg_ref, wsum_ref, lab_ref, lb_ref, lbb_ref,
                   ua_ref, ub_ref, b_ref, emb_ref, sums_ref):
    xb = x_ref[...]
    selfm = _self_gate(xb, wsum_ref[...], lab_ref[...], lb_ref[...],
                       lbb_ref[...])
    aggr = agg_ref[0] + agg_ref[1] + selfm
    e = jnp.maximum(_dot(xb, ua_ref[...]) + _dot(aggr, ub_ref[...])
                    + b_ref[...], 0.0)
    emb_ref[...] = e

    @pl.when(pl.program_id(0) == 0)
    def _():
        sums_ref[...] = jnp.zeros_like(sums_ref)
    sums_ref[...] += jnp.concatenate(
        [jnp.sum(e, axis=0, keepdims=True),
         jnp.sum(e * e, axis=0, keepdims=True)], axis=0)


def _combine0(x, agg, wsumT, lab, lb, lbb, ua, ub, b):
    return pl.pallas_call(
        _combine0_body,
        grid=(GRID,),
        in_specs=[
            pl.BlockSpec((RB, D), lambda i: (i, 0)),
            pl.BlockSpec((NC, RB, D), lambda i: (0, i, 0)),
            pl.BlockSpec((D, HK), lambda i: (0, 0)),
            pl.BlockSpec((1, HK), lambda i: (0, 0)),
            pl.BlockSpec((1, HK), lambda i: (0, 0)),
            pl.BlockSpec((1, 1), lambda i: (0, 0)),
            pl.BlockSpec((D, D), lambda i: (0, 0)),
            pl.BlockSpec((D, D), lambda i: (0, 0)),
            pl.BlockSpec((1, D), lambda i: (0, 0)),
        ],
        out_specs=[
            pl.BlockSpec((RB, D), lambda i: (i, 0)),
            pl.BlockSpec((2, D), lambda i: (0, 0)),
        ],
        out_shape=[
            jax.ShapeDtypeStruct((N, D), jnp.float32),
            jax.ShapeDtypeStruct((2, D), jnp.float32),
        ],
    )(x, agg, wsumT, lab, lb, lbb, ua, ub, b)


def _normproj_body(emb_ref, sums_ref, gamma_ref, beta_ref,
                   wd_ref, ws_ref, lab_ref, xn_ref, a_ref, b_ref):
    sums = sums_ref[...]
    mean = sums[0:1] * (1.0 / N)
    var = sums[1:2] * (1.0 / N) - mean * mean
    xb = emb_ref[...]
    xn = ((xb - mean) * lax.rsqrt(var + EPS) * gamma_ref[...]
          + beta_ref[...])
    xn_ref[...] = xn
    a_ref[...] = _dot(xn, wd_ref[...]) + lab_ref[...]
    b_ref[...] = _dot(xn, ws_ref[...])


def _normproj(emb, sums, gamma, beta, wdT, wsT, lab):
    return pl.pallas_call(
        _normproj_body,
        grid=(GRID,),
        in_specs=[
            pl.BlockSpec((RB, D), lambda i: (i, 0)),
            pl.BlockSpec((2, D), lambda i: (0, 0)),
            pl.BlockSpec((1, D), lambda i: (0, 0)),
            pl.BlockSpec((1, D), lambda i: (0, 0)),
            pl.BlockSpec((D, HK), lambda i: (0, 0)),
            pl.BlockSpec((D, HK), lambda i: (0, 0)),
            pl.BlockSpec((1, HK), lambda i: (0, 0)),
        ],
        out_specs=[
            pl.BlockSpec((RB, D), lambda i: (i, 0)),
            pl.BlockSpec((RB, HK), lambda i: (i, 0)),
            pl.BlockSpec((RB, HK), lambda i: (i, 0)),
        ],
        out_shape=[
            jax.ShapeDtypeStruct((N, D), jnp.float32),
            jax.ShapeDtypeStruct((N, HK), jnp.float32),
            jax.ShapeDtypeStruct((N, HK), jnp.float32),
        ],
    )(emb, sums, gamma, beta, wdT, wsT, lab)


def _combine1_body(xn_ref, agg_ref, wsum_ref, lab_ref, lb_ref, lbb_ref,
                   ua_ref, ub_ref, b_ref, few_ref, feb_ref, out_ref):
    xb = xn_ref[...]
    selfm = _self_gate(xb, wsum_ref[...], lab_ref[...], lb_ref[...],
                       lbb_ref[...])
    aggr = agg_ref[0] + agg_ref[1] + selfm
    e = jnp.maximum(_dot(xb, ua_ref[...]) + _dot(aggr, ub_ref[...])
                    + b_ref[...], 0.0)
    out_ref[...] = _dot(e, few_ref[...]) + feb_ref[...]


def _combine1(xn, agg, wsumT, lab, lb, lbb, ua, ub, b, fewT, feb):
    return pl.pallas_call(
        _combine1_body,
        grid=(GRID,),
        in_specs=[
            pl.BlockSpec((RB, D), lambda i: (i, 0)),
            pl.BlockSpec((NC, RB, D), lambda i: (0, i, 0)),
            pl.BlockSpec((D, HK), lambda i: (0, 0)),
            pl.BlockSpec((1, HK), lambda i: (0, 0)),
            pl.BlockSpec((1, HK), lambda i: (0, 0)),
            pl.BlockSpec((1, 1), lambda i: (0, 0)),
            pl.BlockSpec((D, D), lambda i: (0, 0)),
            pl.BlockSpec((D, D), lambda i: (0, 0)),
            pl.BlockSpec((1, D), lambda i: (0, 0)),
            pl.BlockSpec((D, EMB), lambda i: (0, 0)),
            pl.BlockSpec((1, EMB), lambda i: (0, 0)),
        ],
        out_specs=pl.BlockSpec((RB, EMB), lambda i: (i, 0)),
        out_shape=jax.ShapeDtypeStruct((N, EMB), jnp.float32),
    )(xn, agg, wsumT, lab, lb, lbb, ua, ub, b, fewT, feb)


def kernel(x, edge_index, edge_attr,
           l0_lin_W, l0_lin_b, l0_linA_W, l0_linA_b, l0_linB_W, l0_linB_b,
           l1_bn_gamma, l1_bn_beta, l1_lin_W, l1_lin_b, l1_linA_W, l1_linA_b,
           l1_linB_W, l1_linB_b, fe_W, fe_b):
    f32 = jnp.float32
    src = edge_index[0].astype(jnp.int32)
    dst = edge_index[1].astype(jnp.int32)
    ea = edge_attr.astype(f32)
    pad = EPADX - E
    zpad_i = jnp.zeros((pad,), jnp.int32)
    epack = jnp.stack([
        jnp.concatenate([src, zpad_i]),
        jnp.concatenate([dst, zpad_i]),
        jnp.concatenate([lax.bitcast_convert_type(ea, jnp.int32), zpad_i]),
    ], axis=0)

    def prep(laW, lab, lbW, lbb):
        wdT = laW[:, :D].T
        wsT = laW[:, D:2 * D].T
        wsumT = wdT + wsT
        cvec = laW[:, 2 * D]
        cl = jnp.stack([cvec, lbW[0],
                        jnp.full((HK,), lbb[0], f32)], axis=0)
        lab2 = lab.reshape(1, HK)
        lb2 = lbW.reshape(1, HK)
        lbb2 = lbb.reshape(1, 1)
        return wdT, wsT, wsumT, cl, lab2, lb2, lbb2

    wdT0, wsT0, wsumT0, cl0, lab0, lb0, lbb0 = prep(
        l0_linA_W, l0_linA_b, l0_linB_W, l0_linB_b)
    wdT1, wsT1, wsumT1, cl1, lab1, lb1, lbb1 = prep(
        l1_linA_W, l1_linA_b, l1_linB_W, l1_linB_b)

    ua0 = l0_lin_W[:, :D].T
    ub0 = l0_lin_W[:, D:].T
    b0 = l0_lin_b.reshape(1, D)
    ua1 = l1_lin_W[:, :D].T
    ub1 = l1_lin_W[:, D:].T
    b1 = l1_lin_b.reshape(1, D)
    fewT = fe_W.T
    feb2 = fe_b.reshape(1, EMB)
    gamma = l1_bn_gamma.reshape(1, D)
    beta = l1_bn_beta.reshape(1, D)

    a0, bvec0 = _proj(x, wdT0, wsT0, lab0)
    agg0 = _edge_aggregate(epack, a0, bvec0, x, cl0)
    emb0, sums = _combine0(x, agg0, wsumT0, lab0, lb0, lbb0, ua0, ub0, b0)

    xn, a1, bvec1 = _normproj(emb0, sums, gamma, beta, wdT1, wsT1, lab1)
    agg1 = _edge_aggregate(epack, a1, bvec1, xn, cl1)
    return _combine1(xn, agg1, wsumT1, lab1, lb1, lbb1, ua1, ub1, b1,
                     fewT, feb2)


# speedup vs baseline: 1.2236x; 1.0211x over previous
import functools

import jax
import jax.numpy as jnp
from jax import lax
from jax.experimental import pallas as pl
from jax.experimental.pallas import tpu as pltpu
from jax.experimental.pallas import tpu_sc as plsc

N = 10000
E = 320000
D = 128
HK = 16
EMB = 64
EPS = 1e-5

NC = 2
NS = 16
NW = NC * NS
CE = 64
EPT = 10240
T0 = 240
T1 = 80
NCHUNK = EPT // CE
EPAD = NW * EPT
NBUF = 4
EPADX = EPAD + 3 * CE
NPAD = 10240
ROWS_PER_TILE = NPAD // NS

RB = 1000
GRID = N // RB


def _edge_body(ep_hbm, a_hbm, b_hbm, x_hbm, cl_hbm,
               out_hbm,
               epk0, epk1, epk2, epk3, ad0, ad1, ad2, ad3,
               bs0, bs1, bs2, bs3, xb0, xb1, xb2, xb3, wv, clv,
               aggr, sem_i0, sem_i1, sem_i2, sem_i3,
               sem_g0, sem_g1, sem_g2, sem_g3,
               sem_s0, sem_s1, sem_s2, sem_s3):
    c = lax.axis_index("c")
    s = lax.axis_index("s")
    ebase = jnp.where(c == 0, s * (T0 * CE), NS * (T0 * CE) + s * (T1 * CE))
    nquad = jnp.where(c == 0, T0 // NBUF, T1 // NBUF)

    epk = (epk0, epk1, epk2, epk3)
    ad = (ad0, ad1, ad2, ad3)
    bs = (bs0, bs1, bs2, bs3)
    xb = (xb0, xb1, xb2, xb3)
    sem_i = (sem_i0, sem_i1, sem_i2, sem_i3)
    sem_g = (sem_g0, sem_g1, sem_g2, sem_g3)
    sem_s = (sem_s0, sem_s1, sem_s2, sem_s3)

    pltpu.sync_copy(cl_hbm, clv)
    cvec = clv[0, :]
    lvec = clv[1, :]
    lbb_s = clv[2, :][0]

    def _zrow(i, carry):
        for r in range(D // 16):
            xb0[i, pl.ds(r * 16, 16)] = jnp.zeros((16,), jnp.float32)
        return carry
    lax.fori_loop(0, CE, _zrow, 0)

    def _zcp(j, carry):
        pltpu.sync_copy(xb0, aggr.at[pl.ds(s * ROWS_PER_TILE + j * CE, CE)])
        return carry
    lax.fori_loop(0, ROWS_PER_TILE // CE, _zcp, 0)
    plsc.subcore_barrier()

    def _issue_idx(i, b):
        pltpu.async_copy(ep_hbm.at[:, pl.ds(ebase + i * CE, CE)],
                         epk[b], sem_i[b])

    def _wait_idx(b):
        pltpu.make_async_copy(ep_hbm.at[:, pl.ds(ebase, CE)],
                              epk[b], sem_i[b]).wait()

    def _issue_gathers(b):
        pltpu.async_copy(a_hbm.at[epk[b].at[1]], ad[b], sem_g[b])
        pltpu.async_copy(b_hbm.at[epk[b].at[0]], bs[b], sem_g[b])
        pltpu.async_copy(x_hbm.at[epk[b].at[0]], xb[b], sem_g[b])

    def _wait_gathers(b):
        pltpu.make_async_copy(a_hbm.at[epk[b].at[1]], ad[b], sem_g[b]).wait()
        pltpu.make_async_copy(b_hbm.at[epk[b].at[0]], bs[b], sem_g[b]).wait()
        pltpu.make_async_copy(x_hbm.at[epk[b].at[0]], xb[b], sem_g[b]).wait()

    def _wait_scatter(b):
        pltpu.make_async_copy(xb[b], aggr.at[epk[b].at[1]], sem_s[b]).wait()

    _issue_idx(0, 0)
    _issue_idx(1, 1)
    _issue_idx(2, 2)
    _wait_idx(0)
    _issue_gathers(0)
    _wait_idx(1)
    _issue_gathers(1)

    def _quad(j, carry):
        for b in range(NBUF):
            i = j * NBUF + b

            @pl.when(i >= 1)
            def _():
                _wait_scatter((b - 1) % NBUF)

            _issue_idx(i + 3, (b + 3) % NBUF)
            _wait_gathers(b)

            def _grp(g, carry2, b=b):
                o = g * 16
                sv = epk[b][0, pl.ds(o, 16)]
                dv = epk[b][1, pl.ds(o, 16)]
                ev = plsc.bitcast(epk[b][2, pl.ds(o, 16)], jnp.float32)
                rows = lax.iota(jnp.int32, 16) + o
                logit = jnp.zeros((16,), jnp.float32) + lbb_s
                for k in range(HK):
                    colk = jnp.full((16,), k, jnp.int32)
                    hk = (plsc.load_gather(ad[b], [rows, colk])
                          + plsc.load_gather(bs[b], [rows, colk])
                          + ev * cvec[k])
                    hk = jnp.maximum(hk, 0.0)
                    logit = logit + hk * lvec[k]
                w = 1.0 / (1.0 + jnp.exp(-logit))
                w = jnp.where(sv != dv, w, 0.0)
                wv[pl.ds(o, 16)] = w
                return carry2
            lax.fori_loop(0, CE // 16, _grp, 0)

            _wait_idx((b + 2) % NBUF)
            _issue_gathers((b + 2) % NBUF)

            def _scl(g, carry2, b=b):
                o = g * 16
                wvec = wv[pl.ds(o, 16)]
                for e in range(16):
                    w_s = wvec[e]
                    for r in range(D // 16):
                        xb[b][o + e, pl.ds(r * 16, 16)] = (
                            xb[b][o + e, pl.ds(r * 16, 16)] * w_s)
                return carry2
            lax.fori_loop(0, CE // 16, _scl, 0)

            pltpu.async_copy(xb[b], aggr.at[epk[b].at[1]], sem_s[b],
                             add=True)
        return carry
    lax.fori_loop(0, nquad, _quad, 0)

    _wait_scatter(NBUF - 1)
    _wait_gathers(0)
    _wait_gathers(1)
    _wait_idx(2)

    plsc.subcore_barrier()
    pltpu.sync_copy(aggr.at[pl.ds(s * ROWS_PER_TILE, ROWS_PER_TILE)],
                    out_hbm.at[c, pl.ds(s * ROWS_PER_TILE, ROWS_PER_TILE)])


_edge_aggregate = functools.partial(
    pl.kernel,
    out_type=jax.ShapeDtypeStruct((NC, NPAD, D), jnp.float32),
    mesh=plsc.VectorSubcoreMesh(core_axis_name="c", subcore_axis_name="s"),
    compiler_params=pltpu.CompilerParams(needs_layout_passes=False,
                                         use_tc_tiling_on_sc=False),
    scratch_types=(
        [pltpu.VMEM((3, CE), jnp.int32)] * NBUF
        + [pltpu.VMEM((CE, HK), jnp.float32)] * NBUF
        + [pltpu.VMEM((CE, HK), jnp.float32)] * NBUF
        + [pltpu.VMEM((CE, D), jnp.float32)] * NBUF
        + [
            pltpu.VMEM((CE,), jnp.float32),
            pltpu.VMEM((3, HK), jnp.float32),
            pltpu.VMEM_SHARED((NPAD, D), jnp.float32),
        ]
        + [pltpu.SemaphoreType.DMA] * 12
    ),
)(_edge_body)


def _dot(a, b):
    return jnp.dot(a, b, preferred_element_type=jnp.float32)


def _proj_body(x_ref, wd_ref, ws_ref, lab_ref, a_ref, b_ref):
    xb = x_ref[...]
    a_ref[...] = _dot(xb, wd_ref[...]) + lab_ref[...]
    b_ref[...] = _dot(xb, ws_ref[...])


def _proj(x, wdT, wsT, lab):
    return pl.pallas_call(
        _proj_body,
        grid=(GRID,),
        in_specs=[
            pl.BlockSpec((RB, D), lambda i: (i, 0)),
            pl.BlockSpec((D, HK), lambda i: (0, 0)),
            pl.BlockSpec((D, HK), lambda i: (0, 0)),
            pl.BlockSpec((1, HK), lambda i: (0, 0)),
        ],
        out_specs=[
            pl.BlockSpec((RB, HK), lambda i: (i, 0)),
            pl.BlockSpec((RB, HK), lambda i: (i, 0)),
        ],
        out_shape=[
            jax.ShapeDtypeStruct((N, HK), jnp.float32),
            jax.ShapeDtypeStruct((N, HK), jnp.float32),
        ],
    )(x, wdT, wsT, lab)


def _self_gate(xb, wsumT, lab, lb, lbb):
    hs = jnp.maximum(_dot(xb, wsumT) + lab, 0.0)
    logit = jnp.sum(hs * lb, axis=1, keepdims=True) + lbb
    return (1.0 / (1.0 + jnp.exp(-logit))) * xb


def _combine0_body(x_ref, agg_ref, wsum_ref, lab_ref, lb_ref, lbb_ref,
                   ua_ref, ub_ref, b_ref, emb_ref, sums_ref):
    xb = x_ref[...]
    selfm = _self_gate(xb, wsum_ref[...], lab_ref[...], lb_ref[...],
                       lbb_ref[...])
    aggr = agg_ref[0] + agg_ref[1] + selfm
    e = jnp.maximum(_dot(xb, ua_ref[...]) + _dot(aggr, ub_ref[...])
                    + b_ref[...], 0.0)
    emb_ref[...] = e

    @pl.when(pl.program_id(0) == 0)
    def _():
        sums_ref[...] = jnp.zeros_like(sums_ref)
    sums_ref[...] += jnp.concatenate(
        [jnp.sum(e, axis=0, keepdims=True),
         jnp.sum(e * e, axis=0, keepdims=True)], axis=0)


def _combine0(x, agg, wsumT, lab, lb, lbb, ua, ub, b):
    return pl.pallas_call(
        _combine0_body,
        grid=(GRID,),
        in_specs=[
            pl.BlockSpec((RB, D), lambda i: (i, 0)),
            pl.BlockSpec((NC, RB, D), lambda i: (0, i, 0)),
            pl.BlockSpec((D, HK), lambda i: (0, 0)),
            pl.BlockSpec((1, HK), lambda i: (0, 0)),
            pl.BlockSpec((1, HK), lambda i: (0, 0)),
            pl.BlockSpec((1, 1), lambda i: (0, 0)),
            pl.BlockSpec((D, D), lambda i: (0, 0)),
            pl.BlockSpec((D, D), lambda i: (0, 0)),
            pl.BlockSpec((1, D), lambda i: (0, 0)),
        ],
        out_specs=[
            pl.BlockSpec((RB, D), lambda i: (i, 0)),
            pl.BlockSpec((2, D), lambda i: (0, 0)),
        ],
        out_shape=[
            jax.ShapeDtypeStruct((N, D), jnp.float32),
            jax.ShapeDtypeStruct((2, D), jnp.float32),
        ],
    )(x, agg, wsumT, lab, lb, lbb, ua, ub, b)


def _normproj_body(emb_ref, sums_ref, gamma_ref, beta_ref,
                   wd_ref, ws_ref, lab_ref, xn_ref, a_ref, b_ref):
    sums = sums_ref[...]
    mean = sums[0:1] * (1.0 / N)
    var = sums[1:2] * (1.0 / N) - mean * mean
    xb = emb_ref[...]
    xn = ((xb - mean) * lax.rsqrt(var + EPS) * gamma_ref[...]
          + beta_ref[...])
    xn_ref[...] = xn
    a_ref[...] = _dot(xn, wd_ref[...]) + lab_ref[...]
    b_ref[...] = _dot(xn, ws_ref[...])


def _normproj(emb, sums, gamma, beta, wdT, wsT, lab):
    return pl.pallas_call(
        _normproj_body,
        grid=(GRID,),
        in_specs=[
            pl.BlockSpec((RB, D), lambda i: (i, 0)),
            pl.BlockSpec((2, D), lambda i: (0, 0)),
            pl.BlockSpec((1, D), lambda i: (0, 0)),
            pl.BlockSpec((1, D), lambda i: (0, 0)),
            pl.BlockSpec((D, HK), lambda i: (0, 0)),
            pl.BlockSpec((D, HK), lambda i: (0, 0)),
            pl.BlockSpec((1, HK), lambda i: (0, 0)),
        ],
        out_specs=[
            pl.BlockSpec((RB, D), lambda i: (i, 0)),
            pl.BlockSpec((RB, HK), lambda i: (i, 0)),
            pl.BlockSpec((RB, HK), lambda i: (i, 0)),
        ],
        out_shape=[
            jax.ShapeDtypeStruct((N, D), jnp.float32),
            jax.ShapeDtypeStruct((N, HK), jnp.float32),
            jax.ShapeDtypeStruct((N, HK), jnp.float32),
        ],
    )(emb, sums, gamma, beta, wdT, wsT, lab)


def _combine1_body(xn_ref, agg_ref, wsum_ref, lab_ref, lb_ref, lbb_ref,
                   ua_ref, ub_ref, b_ref, few_ref, feb_ref, out_ref):
    xb = xn_ref[...]
    selfm = _self_gate(xb, wsum_ref[...], lab_ref[...], lb_ref[...],
                       lbb_ref[...])
    aggr = agg_ref[0] + agg_ref[1] + selfm
    e = jnp.maximum(_dot(xb, ua_ref[...]) + _dot(aggr, ub_ref[...])
                    + b_ref[...], 0.0)
    out_ref[...] = _dot(e, few_ref[...]) + feb_ref[...]


def _combine1(xn, agg, wsumT, lab, lb, lbb, ua, ub, b, fewT, feb):
    return pl.pallas_call(
        _combine1_body,
        grid=(GRID,),
        in_specs=[
            pl.BlockSpec((RB, D), lambda i: (i, 0)),
            pl.BlockSpec((NC, RB, D), lambda i: (0, i, 0)),
            pl.BlockSpec((D, HK), lambda i: (0, 0)),
            pl.BlockSpec((1, HK), lambda i: (0, 0)),
            pl.BlockSpec((1, HK), lambda i: (0, 0)),
            pl.BlockSpec((1, 1), lambda i: (0, 0)),
            pl.BlockSpec((D, D), lambda i: (0, 0)),
            pl.BlockSpec((D, D), lambda i: (0, 0)),
            pl.BlockSpec((1, D), lambda i: (0, 0)),
            pl.BlockSpec((D, EMB), lambda i: (0, 0)),
            pl.BlockSpec((1, EMB), lambda i: (0, 0)),
        ],
        out_specs=pl.BlockSpec((RB, EMB), lambda i: (i, 0)),
        out_shape=jax.ShapeDtypeStruct((N, EMB), jnp.float32),
    )(xn, agg, wsumT, lab, lb, lbb, ua, ub, b, fewT, feb)


def kernel(x, edge_index, edge_attr,
           l0_lin_W, l0_lin_b, l0_linA_W, l0_linA_b, l0_linB_W, l0_linB_b,
           l1_bn_gamma, l1_bn_beta, l1_lin_W, l1_lin_b, l1_linA_W, l1_linA_b,
           l1_linB_W, l1_linB_b, fe_W, fe_b):
    f32 = jnp.float32
    src = edge_index[0].astype(jnp.int32)
    dst = edge_index[1].astype(jnp.int32)
    ea = edge_attr.astype(f32)
    pad = EPADX - E
    zpad_i = jnp.zeros((pad,), jnp.int32)
    epack = jnp.stack([
        jnp.concatenate([src, zpad_i]),
        jnp.concatenate([dst, zpad_i]),
        jnp.concatenate([lax.bitcast_convert_type(ea, jnp.int32), zpad_i]),
    ], axis=0)

    def prep(laW, lab, lbW, lbb):
        wdT = laW[:, :D].T
        wsT = laW[:, D:2 * D].T
        wsumT = wdT + wsT
        cvec = laW[:, 2 * D]
        cl = jnp.stack([cvec, lbW[0],
                        jnp.full((HK,), lbb[0], f32)], axis=0)
        lab2 = lab.reshape(1, HK)
        lb2 = lbW.reshape(1, HK)
        lbb2 = lbb.reshape(1, 1)
        return wdT, wsT, wsumT, cl, lab2, lb2, lbb2

    wdT0, wsT0, wsumT0, cl0, lab0, lb0, lbb0 = prep(
        l0_linA_W, l0_linA_b, l0_linB_W, l0_linB_b)
    wdT1, wsT1, wsumT1, cl1, lab1, lb1, lbb1 = prep(
        l1_linA_W, l1_linA_b, l1_linB_W, l1_linB_b)

    ua0 = l0_lin_W[:, :D].T
    ub0 = l0_lin_W[:, D:].T
    b0 = l0_lin_b.reshape(1, D)
    ua1 = l1_lin_W[:, :D].T
    ub1 = l1_lin_W[:, D:].T
    b1 = l1_lin_b.reshape(1, D)
    fewT = fe_W.T
    feb2 = fe_b.reshape(1, EMB)
    gamma = l1_bn_gamma.reshape(1, D)
    beta = l1_bn_beta.reshape(1, D)

    a0, bvec0 = _proj(x, wdT0, wsT0, lab0)
    agg0 = _edge_aggregate(epack, a0, bvec0, x, cl0)
    emb0, sums = _combine0(x, agg0, wsumT0, lab0, lb0, lbb0, ua0, ub0, b0)

    xn, a1, bvec1 = _normproj(emb0, sums, gamma, beta, wdT1, wsT1, lab1)
    agg1 = _edge_aggregate(epack, a1, bvec1, xn, cl1)
    return _combine1(xn, agg1, wsumT1, lab1, lb1, lbb1, ua1, ub1, b1,
                     fewT, feb2)
